# Initial kernel scaffold; baseline (speedup 1.0000x reference)
import jax
import jax.numpy as jnp
from jax.experimental import pallas as pl


def kernel(triples, ent_emb, rel_emb):
    raise NotImplementedError("write your pallas kernel here")


# speedup vs baseline: 11.7346x; 11.7346x over previous
import functools

import jax
import jax.numpy as jnp
from jax.experimental import pallas as pl
from jax.experimental.pallas import tpu as pltpu

_PI = 3.141592653589793
_B = 32
_D = 16


def _prep_body(triples_ref, entT_ref, rel2_ref, ent_hbm, rel_hbm,
               sinA_ref, cosA_ref, scale_ref, heads, rels, sem):
    for b in range(_B):
        hi = triples_ref[b, 0]
        ri = triples_ref[b, 1]
        pltpu.make_async_copy(ent_hbm.at[pl.ds(hi, 1), :],
                              heads.at[pl.ds(b, 1), :], sem).start()
        pltpu.make_async_copy(rel_hbm.at[pl.ds(ri, 1), :],
                              rels.at[pl.ds(b, 1), :], sem).start()

    s_ent = _PI / jnp.max(jnp.abs(entT_ref[...]))
    s_rel = _PI / jnp.max(jnp.abs(rel2_ref[...]))
    scale_ref[0, 0] = s_ent

    for b in range(_B):
        pltpu.make_async_copy(ent_hbm.at[pl.ds(0, 1), :],
                              heads.at[pl.ds(b, 1), :], sem).wait()
        pltpu.make_async_copy(rel_hbm.at[pl.ds(0, 1), :],
                              rels.at[pl.ds(b, 1), :], sem).wait()

    a = heads[...] * s_ent + rels[...] * s_rel
    sinA_ref[...] = jnp.sin(a)
    cosA_ref[...] = jnp.cos(a)


def _sweep_body(scale_ref, sinA_ref, cosA_ref, entT_ref, out_ref):
    s = scale_ref[0, 0]
    p = entT_ref[...] * s
    sp = jnp.sin(p)
    cp = jnp.cos(p)
    dn = (((1,), (0,)), ((), ()))
    out_ref[...] = (
        jax.lax.dot_general(cosA_ref[...], sp, dn,
                            preferred_element_type=jnp.float32)
        - jax.lax.dot_general(sinA_ref[...], cp, dn,
                              preferred_element_type=jnp.float32)
    )


@jax.jit
def kernel(triples, ent_emb, rel_emb):
    num_ent, d = ent_emb.shape
    entT = ent_emb.T
    rel2 = rel_emb.reshape(-1, 128)
    triples = triples.astype(jnp.int32)

    sinA, cosA, scale = pl.pallas_call(
        _prep_body,
        grid=(),
        in_specs=[
            pl.BlockSpec(memory_space=pltpu.SMEM),
            pl.BlockSpec(memory_space=pltpu.VMEM),
            pl.BlockSpec(memory_space=pltpu.VMEM),
            pl.BlockSpec(memory_space=pl.ANY),
            pl.BlockSpec(memory_space=pl.ANY),
        ],
        out_specs=[
            pl.BlockSpec(memory_space=pltpu.VMEM),
            pl.BlockSpec(memory_space=pltpu.VMEM),
            pl.BlockSpec(memory_space=pltpu.SMEM),
        ],
        out_shape=[
            jax.ShapeDtypeStruct((_B, _D), jnp.float32),
            jax.ShapeDtypeStruct((_B, _D), jnp.float32),
            jax.ShapeDtypeStruct((1, 1), jnp.float32),
        ],
        scratch_shapes=[
            pltpu.VMEM((_B, _D), jnp.float32),
            pltpu.VMEM((_B, _D), jnp.float32),
            pltpu.SemaphoreType.DMA,
        ],
    )(triples, entT, rel2, ent_emb, rel_emb)

    bn = 4096
    grid = (num_ent + bn - 1) // bn
    out = pl.pallas_call(
        _sweep_body,
        grid=(grid,),
        in_specs=[
            pl.BlockSpec(memory_space=pltpu.SMEM),
            pl.BlockSpec((_B, _D), lambda i: (0, 0)),
            pl.BlockSpec((_B, _D), lambda i: (0, 0)),
            pl.BlockSpec((_D, bn), lambda i: (0, i)),
        ],
        out_specs=pl.BlockSpec((_B, bn), lambda i: (0, i)),
        out_shape=jax.ShapeDtypeStruct((_B, num_ent), jnp.float32),
    )(scale, sinA, cosA, entT)
    return out
